# Initial kernel scaffold; baseline (speedup 1.0000x reference)
import jax
import jax.numpy as jnp
from jax.experimental import pallas as pl


def kernel(rate_edges, trust_edges, pos_edges, neg_edges, pq_user, pq_item, yw_user, yw_item, b_user, b_item):
    raise NotImplementedError("write your pallas kernel here")


# speedup vs baseline: 3.5910x; 3.5910x over previous
import functools

import jax
import jax.numpy as jnp
from jax import lax
from jax.experimental import pallas as pl
from jax.experimental.pallas import tpu as pltpu
from jax.experimental.pallas import tpu_sc as plsc

N_USERS = 10000
N_ITEMS = 10000
DIM = 128
LAM = 0.5
GB = 3.5

NC = 2
NS = 16
NW = NC * NS
LANES = 16

R_PAD = 10112
STRIPE = R_PAD // NS
CHUNK = 128
PEND = 272

_cp = pltpu.CompilerParams(needs_layout_passes=False)
_mesh = plsc.VectorSubcoreMesh(core_axis_name="c", subcore_axis_name="s")


NSUB = 8


def _make_deg_kernel(ea_tot):
    ta = ea_tot // NW
    n_chunks = ta // CHUNK

    @functools.partial(
        pl.kernel,
        out_type=jax.ShapeDtypeStruct((NW, R_PAD), jnp.float32),
        mesh=_mesh,
        compiler_params=_cp,
        scratch_types=[
            pltpu.VMEM((CHUNK,), jnp.int32),
            pltpu.VMEM((NSUB * R_PAD,), jnp.float32),
        ],
    )
    def deg_kernel(dst_hbm, out_hbm, idx_v, hist_v):
        c = lax.axis_index("c")
        s = lax.axis_index("s")
        w = s * NC + c

        zeros16 = jnp.zeros((LANES,), jnp.float32)
        ones16 = jnp.ones((LANES,), jnp.float32)
        lane = lax.iota(jnp.int32, LANES)
        lo_mask = lane < NSUB
        sub_base = jnp.where(lo_mask, lane, lane - NSUB) * R_PAD

        def fill_zeros(i, carry):
            hist_v[pl.ds(i * LANES, LANES)] = zeros16
            return carry

        lax.fori_loop(0, NSUB * R_PAD // LANES, fill_zeros, 0)

        def step(k, carry):
            base = w * ta + k * CHUNK
            pltpu.sync_copy(dst_hbm.at[pl.ds(base, CHUNK)], idx_v)
            for g in range(CHUNK // LANES):
                idx = idx_v[pl.ds(g * LANES, LANES)] + sub_base
                plsc.addupdate_scatter(hist_v, [idx], ones16, mask=lo_mask)
                plsc.addupdate_scatter(hist_v, [idx], ones16, mask=~lo_mask)
            return carry

        lax.fori_loop(0, n_chunks, step, 0)

        def red(i, carry):
            acc = hist_v[pl.ds(i * LANES, LANES)]
            for j in range(1, NSUB):
                acc = acc + hist_v[pl.ds(j * R_PAD + i * LANES, LANES)]
            hist_v[pl.ds(i * LANES, LANES)] = acc
            return carry

        lax.fori_loop(0, R_PAD // LANES, red, 0)
        pltpu.sync_copy(hist_v.at[pl.ds(0, R_PAD)], out_hbm.at[w])

    return deg_kernel


def _make_agg_kernel(ea_tot):
    half = ea_tot // NC
    n_chunks = half // CHUNK

    @functools.partial(
        pl.kernel,
        out_type=jax.ShapeDtypeStruct((NC * R_PAD * DIM,), jnp.float32),
        mesh=_mesh,
        compiler_params=_cp,
        scratch_types=[
            pltpu.VMEM((CHUNK,), jnp.int32),
            pltpu.VMEM((CHUNK,), jnp.int32),
            pltpu.VMEM((PEND,), jnp.int32),
            pltpu.VMEM((PEND,), jnp.int32),
            pltpu.VMEM((CHUNK,), jnp.int32),
            pltpu.VMEM((CHUNK, DIM), jnp.float32),
            pltpu.VMEM((STRIPE * DIM,), jnp.float32),
            pltpu.SemaphoreType.DMA,
        ],
    )
    def agg_kernel(src_hbm, dst_hbm, tab_hbm, out_hbm,
                   sidx_v, didx_v, pend_d, pend_l, gidx_v, rows_v,
                   acc_v, sem):
        c = lax.axis_index("c")
        s = lax.axis_index("s")
        lo = s * STRIPE

        zeros16 = jnp.zeros((LANES,), jnp.float32)
        lane = lax.iota(jnp.int32, LANES)

        def fill_zeros(i, carry):
            acc_v[pl.ds(i * LANES, LANES)] = zeros16
            return carry

        lax.fori_loop(0, STRIPE * DIM // LANES, fill_zeros, 0)

        def fire():
            for g in range(CHUNK // LANES):
                gidx_v[pl.ds(g * LANES, LANES)] = (
                    pend_d[pl.ds(g * LANES, LANES)])
            pltpu.async_copy(tab_hbm.at[gidx_v], rows_v, sem).wait()

            def edge(e, carry2):
                l_splat = plsc.load_gather(
                    pend_l, [jnp.full((LANES,), 0, jnp.int32) + e])
                base = l_splat * DIM + lane
                for j in range(DIM // LANES):
                    plsc.addupdate_scatter(
                        acc_v, [base + j * LANES],
                        rows_v[e, pl.ds(j * LANES, LANES)])
                return carry2

            lax.fori_loop(0, CHUNK, edge, 0)
            for g in range(CHUNK // LANES):
                pend_d[pl.ds(g * LANES, LANES)] = (
                    pend_d[pl.ds(CHUNK + g * LANES, LANES)])
                pend_l[pl.ds(g * LANES, LANES)] = (
                    pend_l[pl.ds(CHUNK + g * LANES, LANES)])

        def step(k, cnt):
            base = c * half + k * CHUNK
            pltpu.sync_copy(src_hbm.at[pl.ds(base, CHUNK)], sidx_v)
            pltpu.sync_copy(dst_hbm.at[pl.ds(base, CHUNK)], didx_v)
            for g in range(CHUNK // LANES):
                sv = sidx_v[pl.ds(g * LANES, LANES)]
                dv = didx_v[pl.ds(g * LANES, LANES)]
                m = (sv >= lo) & (sv < lo + STRIPE)
                plsc.store_compressed(pend_d.at[pl.ds(cnt, LANES)], dv, mask=m)
                plsc.store_compressed(
                    pend_l.at[pl.ds(cnt, LANES)], sv - lo, mask=m)
                npop = plsc.all_reduce_population_count(m)
                cnt = cnt + npop[0]

            def do_fire(cnt_in):
                fire()
                return cnt_in - CHUNK

            cnt = lax.cond(cnt >= CHUNK, do_fire, lambda x: x, cnt)
            return cnt

        cnt = lax.fori_loop(0, n_chunks, step, jnp.int32(0))

        for g in range(CHUNK // LANES):
            pos = g * LANES + lane
            dv = pend_d[pl.ds(g * LANES, LANES)]
            lv = pend_l[pl.ds(g * LANES, LANES)]
            keep = pos < cnt
            pend_d[pl.ds(g * LANES, LANES)] = jnp.where(
                keep, dv, N_ITEMS + lane)
            pend_l[pl.ds(g * LANES, LANES)] = jnp.where(keep, lv, 0)
        fire()

        pltpu.sync_copy(
            acc_v,
            out_hbm.at[pl.ds((c * R_PAD + s * STRIPE) * DIM, STRIPE * DIM)])

    return agg_kernel


def _make_score_kernel(es_tot):
    te = es_tot // NW
    n_chunks = te // CHUNK
    groups = CHUNK // LANES

    @functools.partial(
        pl.kernel,
        out_type=jax.ShapeDtypeStruct((es_tot,), jnp.float32),
        mesh=_mesh,
        compiler_params=_cp,
        scratch_types=[
            pltpu.VMEM((CHUNK,), jnp.int32),
            pltpu.VMEM((CHUNK,), jnp.int32),
            pltpu.VMEM((CHUNK, DIM), jnp.float32),
            pltpu.VMEM((CHUNK, DIM), jnp.float32),
            pltpu.VMEM((CHUNK * LANES,), jnp.float32),
            pltpu.VMEM((CHUNK,), jnp.float32),
            pltpu.VMEM((N_USERS,), jnp.float32),
            pltpu.VMEM((N_ITEMS,), jnp.float32),
            pltpu.SemaphoreType.DMA,
            pltpu.SemaphoreType.DMA,
        ],
    )
    def score_kernel(s_hbm, d_hbm, hu_hbm, hi_hbm, bu_hbm, bi_hbm, out_hbm,
                     sidx_v, didx_v, rows_u, rows_i, pr_v, dots_v,
                     bu_v, bi_v, sem_u, sem_i):
        c = lax.axis_index("c")
        s = lax.axis_index("s")
        w = s * NC + c

        pltpu.sync_copy(bu_hbm, bu_v)
        pltpu.sync_copy(bi_hbm, bi_v)

        lane_iota = lax.iota(jnp.int32, LANES)

        def step(k, carry):
            base = w * te + k * CHUNK
            pltpu.sync_copy(s_hbm.at[pl.ds(base, CHUNK)], sidx_v)
            pltpu.sync_copy(d_hbm.at[pl.ds(base, CHUNK)], didx_v)
            cu = pltpu.async_copy(hu_hbm.at[sidx_v], rows_u, sem_u)
            ci = pltpu.async_copy(hi_hbm.at[didx_v], rows_i, sem_i)
            cu.wait()
            ci.wait()

            def edge(e, carry2):
                acc = (rows_u[e, pl.ds(0, LANES)]
                       * rows_i[e, pl.ds(0, LANES)])
                for j in range(1, DIM // LANES):
                    acc = acc + (rows_u[e, pl.ds(j * LANES, LANES)]
                                 * rows_i[e, pl.ds(j * LANES, LANES)])
                pr_v[pl.ds(e * LANES, LANES)] = acc
                return carry2

            lax.fori_loop(0, CHUNK, edge, 0)

            def group(g, carry2):
                flat = (g * LANES + lane_iota) * LANES
                tot = jnp.zeros((LANES,), jnp.float32)
                for kk in range(LANES):
                    tot = tot + plsc.load_gather(pr_v, [flat + kk])
                su = sidx_v[pl.ds(g * LANES, LANES)]
                du = didx_v[pl.ds(g * LANES, LANES)]
                bu = plsc.load_gather(bu_v, [su])
                bi = plsc.load_gather(bi_v, [du])
                dots_v[pl.ds(g * LANES, LANES)] = tot + bu + bi + GB
                return carry2

            lax.fori_loop(0, groups, group, 0)
            pltpu.sync_copy(dots_v, out_hbm.at[pl.ds(base, CHUNK)])
            return carry

        lax.fori_loop(0, n_chunks, step, 0)

    return score_kernel


def _normalize_tc(degs, ywp):
    def body(degs_ref, ywp_ref, out_ref):
        deg = jnp.sum(degs_ref[...], axis=0)[:, None]
        inv = 1.0 / jnp.clip(deg, 1.0, None)
        out_ref[...] = ywp_ref[...] * inv

    return pl.pallas_call(
        body,
        out_shape=jax.ShapeDtypeStruct((R_PAD, DIM), jnp.float32),
    )(degs, ywp)


def _combine_tc(agg, pq_user):
    def body(agg_ref, pq_ref, out_ref):
        out_ref[...] = (agg_ref[0:N_USERS, :]
                        + agg_ref[R_PAD:R_PAD + N_USERS, :]
                        + pq_ref[...])

    return pl.pallas_call(
        body,
        out_shape=jax.ShapeDtypeStruct((N_USERS, DIM), jnp.float32),
    )(agg, pq_user)


def _reg_tc(pq_user, pq_item, yw_item, b_user, b_item, mean_deg):
    scale = LAM * mean_deg

    def body(pqu_ref, pqi_ref, ywi_ref, bu_ref, bi_ref, out_ref):
        npq_u = jnp.sqrt(jnp.sum(pqu_ref[...] ** 2))
        npq_i = jnp.sqrt(jnp.sum(pqi_ref[...] ** 2))
        nyw_i = jnp.sqrt(jnp.sum(ywi_ref[...] ** 2))
        nb_u = jnp.sqrt(jnp.sum(bu_ref[...] ** 2))
        nb_i = jnp.sqrt(jnp.sum(bi_ref[...] ** 2))
        out_ref[0, 0] = scale * (nb_u + npq_u) + scale * (nb_i + npq_i + nyw_i)

    return pl.pallas_call(
        body,
        out_shape=jax.ShapeDtypeStruct((1, 1), jnp.float32),
        out_specs=pl.BlockSpec(memory_space=pltpu.SMEM),
    )(pq_user, pq_item, yw_item, b_user, b_item)


def _pad_to(n, m):
    return ((n + m - 1) // m) * m


def kernel(rate_edges, trust_edges, pos_edges, neg_edges,
           pq_user, pq_item, yw_user, yw_item, b_user, b_item):
    del trust_edges, yw_user
    e_rate = rate_edges.shape[1]
    e_pred = pos_edges.shape[1]

    ea_tot = _pad_to((e_rate + NW - 1) // NW, CHUNK) * NW
    es_tot = _pad_to((2 * e_pred + NW - 1) // NW, CHUNK) * NW

    pad_a = ea_tot - e_rate
    junk = (jnp.arange(pad_a, dtype=jnp.int32) % (R_PAD - N_USERS)) + N_USERS
    src_pad = jnp.concatenate([rate_edges[0], junk])
    dst_pad = jnp.concatenate([rate_edges[1], junk])

    pad_s = es_tot - 2 * e_pred
    pad_idx = jnp.arange(pad_s, dtype=jnp.int32) % N_USERS
    s_all = jnp.concatenate([pos_edges[0], neg_edges[0], pad_idx])
    d_all = jnp.concatenate([pos_edges[1], neg_edges[1], pad_idx])

    ywp = jnp.concatenate(
        [yw_item, jnp.zeros((R_PAD - N_ITEMS, DIM), jnp.float32)])

    degs = _make_deg_kernel(ea_tot)(dst_pad)
    ywn = _normalize_tc(degs, ywp)
    agg = _make_agg_kernel(ea_tot)(src_pad, dst_pad, ywn)
    h_u = _combine_tc(agg.reshape(NC * R_PAD, DIM), pq_user)

    scores = _make_score_kernel(es_tot)(
        s_all, d_all, h_u, pq_item,
        b_user.reshape(N_USERS), b_item.reshape(N_ITEMS))

    mean_deg = e_rate / N_USERS
    reg = _reg_tc(pq_user, pq_item, yw_item, b_user, b_item, mean_deg)

    pos_score = scores[:e_pred, None]
    neg_score = scores[e_pred:2 * e_pred, None]
    return pos_score, neg_score, reg[0, 0]


# speedup vs baseline: 7.4333x; 2.0700x over previous
import functools

import jax
import jax.numpy as jnp
from jax import lax
from jax.experimental import pallas as pl
from jax.experimental.pallas import tpu as pltpu
from jax.experimental.pallas import tpu_sc as plsc

N_USERS = 10000
N_ITEMS = 10000
DIM = 128
LAM = 0.5
GB = 3.5

NC = 2
NS = 16
NW = NC * NS
LANES = 16

R_PAD = 10112
STRIPE = R_PAD // NS
CHUNK = 128
PEND = 272

_cp = pltpu.CompilerParams(needs_layout_passes=False)
_mesh = plsc.VectorSubcoreMesh(core_axis_name="c", subcore_axis_name="s")


NSUB = 8


def _make_deg_kernel(ea_tot):
    ta = ea_tot // NW
    n_chunks = ta // CHUNK

    @functools.partial(
        pl.kernel,
        out_type=jax.ShapeDtypeStruct((NW, R_PAD), jnp.float32),
        mesh=_mesh,
        compiler_params=_cp,
        scratch_types=[
            pltpu.VMEM((ta,), jnp.int32),
            pltpu.VMEM((NSUB * R_PAD,), jnp.float32),
        ],
    )
    def deg_kernel(dst_hbm, out_hbm, idx_v, hist_v):
        c = lax.axis_index("c")
        s = lax.axis_index("s")
        w = s * NC + c

        zeros16 = jnp.zeros((LANES,), jnp.float32)
        ones16 = jnp.ones((LANES,), jnp.float32)
        lane = lax.iota(jnp.int32, LANES)
        lo_mask = lane < NSUB
        sub_base = jnp.where(lo_mask, lane, lane - NSUB) * R_PAD

        def fill_zeros(i, carry):
            hist_v[pl.ds(i * LANES, LANES)] = zeros16
            return carry

        lax.fori_loop(0, NSUB * R_PAD // LANES, fill_zeros, 0)

        pltpu.sync_copy(dst_hbm.at[pl.ds(w * ta, ta)], idx_v)

        def step(k, carry):
            idx = idx_v[pl.ds(k * LANES, LANES)] + sub_base
            plsc.addupdate_scatter(hist_v, [idx], ones16, mask=lo_mask)
            plsc.addupdate_scatter(hist_v, [idx], ones16, mask=~lo_mask)
            return carry

        lax.fori_loop(0, ta // LANES, step, 0)

        def red(i, carry):
            acc = hist_v[pl.ds(i * LANES, LANES)]
            for j in range(1, NSUB):
                acc = acc + hist_v[pl.ds(j * R_PAD + i * LANES, LANES)]
            hist_v[pl.ds(i * LANES, LANES)] = acc
            return carry

        lax.fori_loop(0, R_PAD // LANES, red, 0)
        pltpu.sync_copy(hist_v.at[pl.ds(0, R_PAD)], out_hbm.at[w])

    return deg_kernel


MACRO = 2048


def _make_agg_kernel(ea_tot):
    half = ea_tot // NC
    n_macro = half // MACRO
    assert half % MACRO == 0

    @functools.partial(
        pl.kernel,
        out_type=jax.ShapeDtypeStruct((NC * R_PAD * DIM,), jnp.float32),
        mesh=_mesh,
        compiler_params=_cp,
        scratch_types=[
            pltpu.VMEM((MACRO,), jnp.int32),
            pltpu.VMEM((MACRO,), jnp.int32),
            pltpu.VMEM((PEND,), jnp.int32),
            pltpu.VMEM((PEND,), jnp.int32),
            pltpu.VMEM((CHUNK,), jnp.int32),
            pltpu.VMEM((CHUNK, DIM), jnp.float32),
            pltpu.VMEM((STRIPE * DIM,), jnp.float32),
            pltpu.SemaphoreType.DMA,
        ],
    )
    def agg_kernel(src_hbm, dst_hbm, tab_hbm, out_hbm,
                   sidx_v, didx_v, pend_d, pend_l, gidx_v, rows_v,
                   acc_v, sem):
        c = lax.axis_index("c")
        s = lax.axis_index("s")
        lo = s * STRIPE

        zeros16 = jnp.zeros((LANES,), jnp.float32)
        lane = lax.iota(jnp.int32, LANES)

        def fill_zeros(i, carry):
            acc_v[pl.ds(i * LANES, LANES)] = zeros16
            return carry

        lax.fori_loop(0, STRIPE * DIM // LANES, fill_zeros, 0)

        def fire():
            for g in range(CHUNK // LANES):
                gidx_v[pl.ds(g * LANES, LANES)] = (
                    pend_d[pl.ds(g * LANES, LANES)])
            pltpu.async_copy(tab_hbm.at[gidx_v], rows_v, sem).wait()

            def edge(e, carry2):
                l_splat = plsc.load_gather(
                    pend_l, [jnp.full((LANES,), 0, jnp.int32) + e])
                base = l_splat * DIM + lane
                for j in range(DIM // LANES):
                    plsc.addupdate_scatter(
                        acc_v, [base + j * LANES],
                        rows_v[e, pl.ds(j * LANES, LANES)])
                return carry2

            lax.fori_loop(0, CHUNK, edge, 0)
            for g in range(CHUNK // LANES):
                pend_d[pl.ds(g * LANES, LANES)] = (
                    pend_d[pl.ds(CHUNK + g * LANES, LANES)])
                pend_l[pl.ds(g * LANES, LANES)] = (
                    pend_l[pl.ds(CHUNK + g * LANES, LANES)])

        def step(k, cnt):
            base = c * half + k * MACRO
            pltpu.sync_copy(src_hbm.at[pl.ds(base, MACRO)], sidx_v)
            pltpu.sync_copy(dst_hbm.at[pl.ds(base, MACRO)], didx_v)

            def sub(q, cnt_q):
                for g in range(CHUNK // LANES):
                    o = q * CHUNK + g * LANES
                    sv = sidx_v[pl.ds(o, LANES)]
                    dv = didx_v[pl.ds(o, LANES)]
                    m = (sv >= lo) & (sv < lo + STRIPE)
                    plsc.store_compressed(
                        pend_d.at[pl.ds(cnt_q, LANES)], dv, mask=m)
                    plsc.store_compressed(
                        pend_l.at[pl.ds(cnt_q, LANES)], sv - lo, mask=m)
                    npop = plsc.all_reduce_population_count(m)
                    cnt_q = cnt_q + npop[0]

                def do_fire(cnt_in):
                    fire()
                    return cnt_in - CHUNK

                cnt_q = lax.cond(cnt_q >= CHUNK, do_fire, lambda x: x, cnt_q)
                return cnt_q

            return lax.fori_loop(0, MACRO // CHUNK, sub, cnt)

        cnt = lax.fori_loop(0, n_macro, step, jnp.int32(0))

        for g in range(CHUNK // LANES):
            pos = g * LANES + lane
            dv = pend_d[pl.ds(g * LANES, LANES)]
            lv = pend_l[pl.ds(g * LANES, LANES)]
            keep = pos < cnt
            pend_d[pl.ds(g * LANES, LANES)] = jnp.where(
                keep, dv, N_ITEMS + lane)
            pend_l[pl.ds(g * LANES, LANES)] = jnp.where(keep, lv, 0)
        fire()

        pltpu.sync_copy(
            acc_v,
            out_hbm.at[pl.ds((c * R_PAD + s * STRIPE) * DIM, STRIPE * DIM)])

    return agg_kernel


def _make_score_kernel(es_tot):
    te = es_tot // NW
    n_chunks = te // CHUNK
    groups = CHUNK // LANES

    @functools.partial(
        pl.kernel,
        out_type=jax.ShapeDtypeStruct((es_tot,), jnp.float32),
        mesh=_mesh,
        compiler_params=_cp,
        scratch_types=[
            pltpu.VMEM((te,), jnp.int32),
            pltpu.VMEM((te,), jnp.int32),
            pltpu.VMEM((CHUNK,), jnp.int32),
            pltpu.VMEM((CHUNK,), jnp.int32),
            pltpu.VMEM((CHUNK, DIM), jnp.float32),
            pltpu.VMEM((CHUNK, DIM), jnp.float32),
            pltpu.VMEM((CHUNK * LANES,), jnp.float32),
            pltpu.VMEM((CHUNK,), jnp.float32),
            pltpu.VMEM((N_USERS,), jnp.float32),
            pltpu.VMEM((N_ITEMS,), jnp.float32),
            pltpu.SemaphoreType.DMA,
            pltpu.SemaphoreType.DMA,
        ],
    )
    def score_kernel(s_hbm, d_hbm, hu_hbm, hi_hbm, bu_hbm, bi_hbm, out_hbm,
                     sall_v, dall_v, sidx_v, didx_v, rows_u, rows_i,
                     pr_v, dots_v, bu_v, bi_v, sem_u, sem_i):
        c = lax.axis_index("c")
        s = lax.axis_index("s")
        w = s * NC + c

        pltpu.sync_copy(bu_hbm, bu_v)
        pltpu.sync_copy(bi_hbm, bi_v)
        pltpu.sync_copy(s_hbm.at[pl.ds(w * te, te)], sall_v)
        pltpu.sync_copy(d_hbm.at[pl.ds(w * te, te)], dall_v)

        lane_iota = lax.iota(jnp.int32, LANES)

        def step(k, carry):
            base = w * te + k * CHUNK
            for g in range(CHUNK // LANES):
                sidx_v[pl.ds(g * LANES, LANES)] = (
                    sall_v[pl.ds(k * CHUNK + g * LANES, LANES)])
                didx_v[pl.ds(g * LANES, LANES)] = (
                    dall_v[pl.ds(k * CHUNK + g * LANES, LANES)])
            cu = pltpu.async_copy(hu_hbm.at[sidx_v], rows_u, sem_u)
            ci = pltpu.async_copy(hi_hbm.at[didx_v], rows_i, sem_i)
            cu.wait()
            ci.wait()

            def edge(e, carry2):
                acc = (rows_u[e, pl.ds(0, LANES)]
                       * rows_i[e, pl.ds(0, LANES)])
                for j in range(1, DIM // LANES):
                    acc = acc + (rows_u[e, pl.ds(j * LANES, LANES)]
                                 * rows_i[e, pl.ds(j * LANES, LANES)])
                pr_v[pl.ds(e * LANES, LANES)] = acc
                return carry2

            lax.fori_loop(0, CHUNK, edge, 0)

            def group(g, carry2):
                flat = (g * LANES + lane_iota) * LANES
                tot = jnp.zeros((LANES,), jnp.float32)
                for kk in range(LANES):
                    tot = tot + plsc.load_gather(pr_v, [flat + kk])
                su = sidx_v[pl.ds(g * LANES, LANES)]
                du = didx_v[pl.ds(g * LANES, LANES)]
                bu = plsc.load_gather(bu_v, [su])
                bi = plsc.load_gather(bi_v, [du])
                dots_v[pl.ds(g * LANES, LANES)] = tot + bu + bi + GB
                return carry2

            lax.fori_loop(0, groups, group, 0)
            pltpu.sync_copy(dots_v, out_hbm.at[pl.ds(base, CHUNK)])
            return carry

        lax.fori_loop(0, n_chunks, step, 0)

    return score_kernel


def _normalize_tc(degs, ywp):
    def body(degs_ref, ywp_ref, out_ref):
        deg = jnp.sum(degs_ref[...], axis=0)[:, None]
        inv = 1.0 / jnp.clip(deg, 1.0, None)
        out_ref[...] = ywp_ref[...] * inv

    return pl.pallas_call(
        body,
        out_shape=jax.ShapeDtypeStruct((R_PAD, DIM), jnp.float32),
    )(degs, ywp)


def _combine_tc(agg, pq_user):
    def body(agg_ref, pq_ref, out_ref):
        out_ref[...] = (agg_ref[0:N_USERS, :]
                        + agg_ref[R_PAD:R_PAD + N_USERS, :]
                        + pq_ref[...])

    return pl.pallas_call(
        body,
        out_shape=jax.ShapeDtypeStruct((N_USERS, DIM), jnp.float32),
    )(agg, pq_user)


def _reg_tc(pq_user, pq_item, yw_item, b_user, b_item, mean_deg):
    scale = LAM * mean_deg

    def body(pqu_ref, pqi_ref, ywi_ref, bu_ref, bi_ref, out_ref):
        npq_u = jnp.sqrt(jnp.sum(pqu_ref[...] ** 2))
        npq_i = jnp.sqrt(jnp.sum(pqi_ref[...] ** 2))
        nyw_i = jnp.sqrt(jnp.sum(ywi_ref[...] ** 2))
        nb_u = jnp.sqrt(jnp.sum(bu_ref[...] ** 2))
        nb_i = jnp.sqrt(jnp.sum(bi_ref[...] ** 2))
        out_ref[0, 0] = scale * (nb_u + npq_u) + scale * (nb_i + npq_i + nyw_i)

    return pl.pallas_call(
        body,
        out_shape=jax.ShapeDtypeStruct((1, 1), jnp.float32),
        out_specs=pl.BlockSpec(memory_space=pltpu.SMEM),
    )(pq_user, pq_item, yw_item, b_user, b_item)


def _pad_to(n, m):
    return ((n + m - 1) // m) * m


def kernel(rate_edges, trust_edges, pos_edges, neg_edges,
           pq_user, pq_item, yw_user, yw_item, b_user, b_item):
    del trust_edges, yw_user
    e_rate = rate_edges.shape[1]
    e_pred = pos_edges.shape[1]

    ea_tot = _pad_to((e_rate + NW - 1) // NW, CHUNK) * NW
    es_tot = _pad_to((2 * e_pred + NW - 1) // NW, CHUNK) * NW

    pad_a = ea_tot - e_rate
    junk = (jnp.arange(pad_a, dtype=jnp.int32) % (R_PAD - N_USERS)) + N_USERS
    src_pad = jnp.concatenate([rate_edges[0], junk])
    dst_pad = jnp.concatenate([rate_edges[1], junk])

    pad_s = es_tot - 2 * e_pred
    pad_idx = jnp.arange(pad_s, dtype=jnp.int32) % N_USERS
    s_all = jnp.concatenate([pos_edges[0], neg_edges[0], pad_idx])
    d_all = jnp.concatenate([pos_edges[1], neg_edges[1], pad_idx])

    ywp = jnp.concatenate(
        [yw_item, jnp.zeros((R_PAD - N_ITEMS, DIM), jnp.float32)])

    degs = _make_deg_kernel(ea_tot)(dst_pad)
    ywn = _normalize_tc(degs, ywp)
    agg = _make_agg_kernel(ea_tot)(src_pad, dst_pad, ywn)
    h_u = _combine_tc(agg.reshape(NC * R_PAD, DIM), pq_user)

    scores = _make_score_kernel(es_tot)(
        s_all, d_all, h_u, pq_item,
        b_user.reshape(N_USERS), b_item.reshape(N_ITEMS))

    mean_deg = e_rate / N_USERS
    reg = _reg_tc(pq_user, pq_item, yw_item, b_user, b_item, mean_deg)

    pos_score = scores[:e_pred, None]
    neg_score = scores[e_pred:2 * e_pred, None]
    return pos_score, neg_score, reg[0, 0]


# speedup vs baseline: 7.8353x; 1.0541x over previous
import functools

import jax
import jax.numpy as jnp
from jax import lax
from jax.experimental import pallas as pl
from jax.experimental.pallas import tpu as pltpu
from jax.experimental.pallas import tpu_sc as plsc

N_USERS = 10000
N_ITEMS = 10000
DIM = 128
LAM = 0.5
GB = 3.5

NC = 2
NS = 16
NW = NC * NS
LANES = 16

R_PAD = 10112
STRIPE = R_PAD // NS
CHUNK = 128
PEND = 272

_cp = pltpu.CompilerParams(needs_layout_passes=False)
_mesh = plsc.VectorSubcoreMesh(core_axis_name="c", subcore_axis_name="s")


NSUB = 8


def _make_deg_kernel(ea_tot):
    ta = ea_tot // NW
    n_chunks = ta // CHUNK

    @functools.partial(
        pl.kernel,
        out_type=jax.ShapeDtypeStruct((NW, R_PAD), jnp.float32),
        mesh=_mesh,
        compiler_params=_cp,
        scratch_types=[
            pltpu.VMEM((ta,), jnp.int32),
            pltpu.VMEM((NSUB * R_PAD,), jnp.float32),
        ],
    )
    def deg_kernel(dst_hbm, out_hbm, idx_v, hist_v):
        c = lax.axis_index("c")
        s = lax.axis_index("s")
        w = s * NC + c

        zeros16 = jnp.zeros((LANES,), jnp.float32)
        ones16 = jnp.ones((LANES,), jnp.float32)
        lane = lax.iota(jnp.int32, LANES)
        lo_mask = lane < NSUB
        sub_base = jnp.where(lo_mask, lane, lane - NSUB) * R_PAD

        def fill_zeros(i, carry):
            hist_v[pl.ds(i * LANES, LANES)] = zeros16
            return carry

        lax.fori_loop(0, NSUB * R_PAD // LANES, fill_zeros, 0)

        pltpu.sync_copy(dst_hbm.at[pl.ds(w * ta, ta)], idx_v)

        def step(k, carry):
            idx = idx_v[pl.ds(k * LANES, LANES)] + sub_base
            plsc.addupdate_scatter(hist_v, [idx], ones16, mask=lo_mask)
            plsc.addupdate_scatter(hist_v, [idx], ones16, mask=~lo_mask)
            return carry

        lax.fori_loop(0, ta // LANES, step, 0)

        def red(i, carry):
            acc = hist_v[pl.ds(i * LANES, LANES)]
            for j in range(1, NSUB):
                acc = acc + hist_v[pl.ds(j * R_PAD + i * LANES, LANES)]
            hist_v[pl.ds(i * LANES, LANES)] = acc
            return carry

        lax.fori_loop(0, R_PAD // LANES, red, 0)
        pltpu.sync_copy(hist_v.at[pl.ds(0, R_PAD)], out_hbm.at[w])

    return deg_kernel


MACRO = 1024
LIST_CAP = 13312
BATCH = 64


def _make_agg_kernel(ea_tot):
    half = ea_tot // NC
    n_macro = half // MACRO
    assert half % MACRO == 0

    @functools.partial(
        pl.kernel,
        out_type=jax.ShapeDtypeStruct((NC * R_PAD * DIM,), jnp.float32),
        mesh=_mesh,
        compiler_params=_cp,
        scratch_types=[
            pltpu.VMEM((MACRO,), jnp.int32),
            pltpu.VMEM((MACRO,), jnp.int32),
            pltpu.VMEM((LIST_CAP,), jnp.int32),
            pltpu.VMEM((LIST_CAP,), jnp.int32),
            pltpu.VMEM((BATCH,), jnp.int32),
            pltpu.VMEM((BATCH,), jnp.int32),
            pltpu.VMEM((2 * BATCH, DIM), jnp.float32),
            pltpu.VMEM((STRIPE * DIM,), jnp.float32),
            pltpu.SemaphoreType.DMA,
            pltpu.SemaphoreType.DMA,
        ],
    )
    def agg_kernel(src_hbm, dst_hbm, tab_hbm, out_hbm,
                   sidx_v, didx_v, list_d, list_l, gidx0, gidx1,
                   rows_v, acc_v, sem0, sem1):
        c = lax.axis_index("c")
        s = lax.axis_index("s")
        lo = s * STRIPE

        zeros16 = jnp.zeros((LANES,), jnp.float32)
        lane = lax.iota(jnp.int32, LANES)

        def fill_zeros(i, carry):
            acc_v[pl.ds(i * LANES, LANES)] = zeros16
            return carry

        lax.fori_loop(0, STRIPE * DIM // LANES, fill_zeros, 0)

        def scan(k, cnt):
            base = c * half + k * MACRO
            pltpu.sync_copy(src_hbm.at[pl.ds(base, MACRO)], sidx_v)
            pltpu.sync_copy(dst_hbm.at[pl.ds(base, MACRO)], didx_v)
            for g in range(MACRO // LANES):
                sv = sidx_v[pl.ds(g * LANES, LANES)]
                dv = didx_v[pl.ds(g * LANES, LANES)]
                m = (sv >= lo) & (sv < lo + STRIPE)
                plsc.store_compressed(
                    list_d.at[pl.ds(cnt, LANES)], dv, mask=m)
                plsc.store_compressed(
                    list_l.at[pl.ds(cnt, LANES)], sv - lo, mask=m)
                npop = plsc.all_reduce_population_count(m)
                cnt = cnt + npop[0]
            return cnt

        cnt = lax.fori_loop(0, n_macro, scan, jnp.int32(0))

        for g in range(BATCH // LANES):
            pos = pl.ds(cnt + g * LANES, LANES)
            list_d[pos] = N_ITEMS + lane
            list_l[pos] = jnp.zeros((LANES,), jnp.int32)
        nb = (cnt + BATCH - 1) // BATCH

        def load_gidx(gidx, b):
            for g in range(BATCH // LANES):
                gidx[pl.ds(g * LANES, LANES)] = (
                    list_d[pl.ds(b * BATCH + g * LANES, LANES)])

        def start(gidx, b, hoff, sem):
            load_gidx(gidx, b)
            pltpu.async_copy(
                tab_hbm.at[gidx], rows_v.at[pl.ds(hoff, BATCH)], sem)

        def wait(gidx, hoff, sem):
            pltpu.make_async_copy(
                tab_hbm.at[gidx], rows_v.at[pl.ds(hoff, BATCH)], sem).wait()

        def rmw(b, hoff):
            def edge(e2, carry2):
                for u in range(2):
                    e = e2 * 2 + u
                    l_splat = plsc.load_gather(
                        list_l, [jnp.full((LANES,), 0, jnp.int32)
                                 + (b * BATCH + e)])
                    base = l_splat * DIM + lane
                    for j in range(DIM // LANES):
                        plsc.addupdate_scatter(
                            acc_v, [base + j * LANES],
                            rows_v[hoff + e, pl.ds(j * LANES, LANES)])
                return carry2

            lax.fori_loop(0, BATCH // 2, edge, 0)

        start(gidx0, 0, 0, sem0)

        def pair(j, carry):
            b0 = 2 * j
            b1 = b0 + 1

            @pl.when(b1 < nb)
            def _():
                start(gidx1, b1, BATCH, sem1)
            wait(gidx0, 0, sem0)
            rmw(b0, 0)

            @pl.when(b1 + 1 < nb)
            def _():
                start(gidx0, b1 + 1, 0, sem0)

            @pl.when(b1 < nb)
            def _():
                wait(gidx1, BATCH, sem1)
                rmw(b1, BATCH)
            return carry

        lax.fori_loop(0, (nb + 1) // 2, pair, 0)

        pltpu.sync_copy(
            acc_v,
            out_hbm.at[pl.ds((c * R_PAD + s * STRIPE) * DIM, STRIPE * DIM)])

    return agg_kernel


def _make_score_kernel(es_tot):
    te = es_tot // NW
    n_chunks = te // CHUNK
    groups = CHUNK // LANES

    @functools.partial(
        pl.kernel,
        out_type=jax.ShapeDtypeStruct((es_tot,), jnp.float32),
        mesh=_mesh,
        compiler_params=_cp,
        scratch_types=[
            pltpu.VMEM((te,), jnp.int32),
            pltpu.VMEM((te,), jnp.int32),
            pltpu.VMEM((CHUNK,), jnp.int32),
            pltpu.VMEM((CHUNK,), jnp.int32),
            pltpu.VMEM((CHUNK, DIM), jnp.float32),
            pltpu.VMEM((CHUNK, DIM), jnp.float32),
            pltpu.VMEM((CHUNK * LANES,), jnp.float32),
            pltpu.VMEM((CHUNK,), jnp.float32),
            pltpu.VMEM((N_USERS,), jnp.float32),
            pltpu.VMEM((N_ITEMS,), jnp.float32),
            pltpu.SemaphoreType.DMA,
            pltpu.SemaphoreType.DMA,
        ],
    )
    def score_kernel(s_hbm, d_hbm, hu_hbm, hi_hbm, bu_hbm, bi_hbm, out_hbm,
                     sall_v, dall_v, sidx_v, didx_v, rows_u, rows_i,
                     pr_v, dots_v, bu_v, bi_v, sem_u, sem_i):
        c = lax.axis_index("c")
        s = lax.axis_index("s")
        w = s * NC + c

        pltpu.sync_copy(bu_hbm, bu_v)
        pltpu.sync_copy(bi_hbm, bi_v)
        pltpu.sync_copy(s_hbm.at[pl.ds(w * te, te)], sall_v)
        pltpu.sync_copy(d_hbm.at[pl.ds(w * te, te)], dall_v)

        lane_iota = lax.iota(jnp.int32, LANES)

        def step(k, carry):
            base = w * te + k * CHUNK
            for g in range(CHUNK // LANES):
                sidx_v[pl.ds(g * LANES, LANES)] = (
                    sall_v[pl.ds(k * CHUNK + g * LANES, LANES)])
                didx_v[pl.ds(g * LANES, LANES)] = (
                    dall_v[pl.ds(k * CHUNK + g * LANES, LANES)])
            cu = pltpu.async_copy(hu_hbm.at[sidx_v], rows_u, sem_u)
            ci = pltpu.async_copy(hi_hbm.at[didx_v], rows_i, sem_i)
            cu.wait()
            ci.wait()

            def edge(e, carry2):
                acc = (rows_u[e, pl.ds(0, LANES)]
                       * rows_i[e, pl.ds(0, LANES)])
                for j in range(1, DIM // LANES):
                    acc = acc + (rows_u[e, pl.ds(j * LANES, LANES)]
                                 * rows_i[e, pl.ds(j * LANES, LANES)])
                pr_v[pl.ds(e * LANES, LANES)] = acc
                return carry2

            lax.fori_loop(0, CHUNK, edge, 0)

            def group(g, carry2):
                flat = (g * LANES + lane_iota) * LANES
                tot = jnp.zeros((LANES,), jnp.float32)
                for kk in range(LANES):
                    tot = tot + plsc.load_gather(pr_v, [flat + kk])
                su = sidx_v[pl.ds(g * LANES, LANES)]
                du = didx_v[pl.ds(g * LANES, LANES)]
                bu = plsc.load_gather(bu_v, [su])
                bi = plsc.load_gather(bi_v, [du])
                dots_v[pl.ds(g * LANES, LANES)] = tot + bu + bi + GB
                return carry2

            lax.fori_loop(0, groups, group, 0)
            pltpu.sync_copy(dots_v, out_hbm.at[pl.ds(base, CHUNK)])
            return carry

        lax.fori_loop(0, n_chunks, step, 0)

    return score_kernel


def _normalize_tc(degs, ywp):
    def body(degs_ref, ywp_ref, out_ref):
        deg = jnp.sum(degs_ref[...], axis=0)[:, None]
        inv = 1.0 / jnp.clip(deg, 1.0, None)
        out_ref[...] = ywp_ref[...] * inv

    return pl.pallas_call(
        body,
        out_shape=jax.ShapeDtypeStruct((R_PAD, DIM), jnp.float32),
    )(degs, ywp)


def _combine_tc(agg, pq_user):
    def body(agg_ref, pq_ref, out_ref):
        out_ref[...] = (agg_ref[0:N_USERS, :]
                        + agg_ref[R_PAD:R_PAD + N_USERS, :]
                        + pq_ref[...])

    return pl.pallas_call(
        body,
        out_shape=jax.ShapeDtypeStruct((N_USERS, DIM), jnp.float32),
    )(agg, pq_user)


def _reg_tc(pq_user, pq_item, yw_item, b_user, b_item, mean_deg):
    scale = LAM * mean_deg

    def body(pqu_ref, pqi_ref, ywi_ref, bu_ref, bi_ref, out_ref):
        npq_u = jnp.sqrt(jnp.sum(pqu_ref[...] ** 2))
        npq_i = jnp.sqrt(jnp.sum(pqi_ref[...] ** 2))
        nyw_i = jnp.sqrt(jnp.sum(ywi_ref[...] ** 2))
        nb_u = jnp.sqrt(jnp.sum(bu_ref[...] ** 2))
        nb_i = jnp.sqrt(jnp.sum(bi_ref[...] ** 2))
        out_ref[0, 0] = scale * (nb_u + npq_u) + scale * (nb_i + npq_i + nyw_i)

    return pl.pallas_call(
        body,
        out_shape=jax.ShapeDtypeStruct((1, 1), jnp.float32),
        out_specs=pl.BlockSpec(memory_space=pltpu.SMEM),
    )(pq_user, pq_item, yw_item, b_user, b_item)


def _pad_to(n, m):
    return ((n + m - 1) // m) * m


def kernel(rate_edges, trust_edges, pos_edges, neg_edges,
           pq_user, pq_item, yw_user, yw_item, b_user, b_item):
    del trust_edges, yw_user
    e_rate = rate_edges.shape[1]
    e_pred = pos_edges.shape[1]

    ea_tot = _pad_to((e_rate + NW - 1) // NW, CHUNK) * NW
    es_tot = _pad_to((2 * e_pred + NW - 1) // NW, CHUNK) * NW

    pad_a = ea_tot - e_rate
    junk = (jnp.arange(pad_a, dtype=jnp.int32) % (R_PAD - N_USERS)) + N_USERS
    src_pad = jnp.concatenate([rate_edges[0], junk])
    dst_pad = jnp.concatenate([rate_edges[1], junk])

    pad_s = es_tot - 2 * e_pred
    pad_idx = jnp.arange(pad_s, dtype=jnp.int32) % N_USERS
    s_all = jnp.concatenate([pos_edges[0], neg_edges[0], pad_idx])
    d_all = jnp.concatenate([pos_edges[1], neg_edges[1], pad_idx])

    ywp = jnp.concatenate(
        [yw_item, jnp.zeros((R_PAD - N_ITEMS, DIM), jnp.float32)])

    degs = _make_deg_kernel(ea_tot)(dst_pad)
    ywn = _normalize_tc(degs, ywp)
    agg = _make_agg_kernel(ea_tot)(src_pad, dst_pad, ywn)
    h_u = _combine_tc(agg.reshape(NC * R_PAD, DIM), pq_user)

    scores = _make_score_kernel(es_tot)(
        s_all, d_all, h_u, pq_item,
        b_user.reshape(N_USERS), b_item.reshape(N_ITEMS))

    mean_deg = e_rate / N_USERS
    reg = _reg_tc(pq_user, pq_item, yw_item, b_user, b_item, mean_deg)

    pos_score = scores[:e_pred, None]
    neg_score = scores[e_pred:2 * e_pred, None]
    return pos_score, neg_score, reg[0, 0]
